# baseline (speedup 1.0000x reference)
import jax
import jax.numpy as jnp
from jax.experimental import pallas as pl
from jax.experimental.pallas import tpu as pltpu

_LAMBDA_MONO = 0.1
_MONO_TOLERANCE = 0.005

_R = 1024
_C = 1024


def _loss_body(p_ref, t_ref, s_ref, out_ref):
    p = p_ref[...]
    t = t_ref[...]
    s = s_ref[...]
    d = p - t
    mse_sum = jnp.sum(d * d)
    within = s[:, 1:] - s[:, :-1]
    cross = s[1:, 0] - s[:-1, _C - 1]
    mono_sum = jnp.sum(jnp.maximum(within - _MONO_TOLERANCE, 0.0)) + jnp.sum(
        jnp.maximum(cross - _MONO_TOLERANCE, 0.0)
    )
    out_ref[0] = mse_sum
    out_ref[1] = mono_sum


def kernel(predictions, targets, cycle_indices):
    n = predictions.shape[0]
    order = jnp.argsort(cycle_indices)
    s = jnp.take(predictions, order).reshape(_R, _C)
    p = predictions.reshape(_R, _C)
    t = targets.reshape(_R, _C)
    sums = pl.pallas_call(
        _loss_body,
        out_shape=jax.ShapeDtypeStruct((2,), jnp.float32),
        out_specs=pl.BlockSpec(memory_space=pltpu.SMEM),
    )(p, t, s)
    loss_mse = sums[0] / n
    loss_mono = sums[1] / (n - 1)
    loss_res = jnp.array(0.0, dtype=jnp.float32)
    total = loss_mse + _LAMBDA_MONO * loss_mono + _LAMBDA_RES_ZERO * loss_res
    return (total, loss_mse, loss_mono, loss_res)


_LAMBDA_RES_ZERO = 0.1
